# baseline (speedup 1.0000x reference)
import dataclasses
import functools

import jax
import jax.numpy as jnp
from jax import lax
from jax.experimental import pallas as pl
from jax.experimental.pallas import tpu as pltpu
from jax.experimental.pallas import tpu_sc as plsc

E = 8
AUX_W = 0.01
Z_W = 0.001
N = 4096
D = 1024
C = 640
LANES = 128
H = 2816
HT = 256
NH = H // HT
EC = E * C

NW = 32
TPW = N // NW
NCHUNK = TPW // 16
DISP_ROWS = EC + NW

_sc_mesh = plsc.VectorSubcoreMesh(core_axis_name="c", subcore_axis_name="s")

_sc_params = pltpu.CompilerParams()
if "needs_layout_passes" in pltpu.CompilerParams.__dataclass_fields__:
    _sc_params = dataclasses.replace(_sc_params, needs_layout_passes=False)


BT = 1024
NTB = N // BT


def _router_body(x_ref, gw_ref, idx_ref, aux_ref, ps_ref, cn_ref, zq_ref):
    t = pl.program_id(0)
    x = x_ref[...]
    gw = gw_ref[...]
    logits = lax.dot_general(
        x.astype(jnp.bfloat16), gw.astype(jnp.bfloat16), (((1,), (1,)), ((), ())),
        preferred_element_type=jnp.float32,
    )
    lane = lax.broadcasted_iota(jnp.int32, (BT, LANES), 1)
    valid = lane < E
    lm = jnp.where(valid, logits, -1e30)
    m = jnp.max(lm, axis=1, keepdims=True)
    top = jnp.min(jnp.where(lm == m, lane, LANES), axis=1, keepdims=True)
    ex = jnp.where(valid, jnp.exp(lm - m), 0.0)
    denom = jnp.sum(ex, axis=1, keepdims=True)
    z = m + jnp.log(denom)
    zsq = jnp.sum(z * z)
    probs_sum = jnp.sum(ex / denom, axis=0, keepdims=True)
    one_hot = (lane == top).astype(jnp.float32)
    counts = jnp.sum(one_hot, axis=0, keepdims=True)
    idx_ref[...] = top

    @pl.when(t == 0)
    def _():
        ps_ref[...] = jnp.zeros((1, LANES), jnp.float32)
        cn_ref[...] = jnp.zeros((1, LANES), jnp.float32)
        zq_ref[0] = 0.0

    ps_ref[...] += probs_sum
    cn_ref[...] += counts
    zq_ref[0] += zsq

    @pl.when(t == NTB - 1)
    def _():
        balance = jnp.sum(ps_ref[...] * cn_ref[...]) * (AUX_W * E / (N * N))
        aux_ref[0, 0] = balance + zq_ref[0] * (Z_W / N)


def _router(x_flat, gw_pad):
    return pl.pallas_call(
        _router_body,
        grid=(NTB,),
        in_specs=[
            pl.BlockSpec((BT, D), lambda t: (t, 0)),
            pl.BlockSpec((LANES, D), lambda t: (0, 0)),
        ],
        out_specs=(
            pl.BlockSpec((BT, 1), lambda t: (t, 0)),
            pl.BlockSpec(memory_space=pltpu.SMEM),
        ),
        out_shape=(
            jax.ShapeDtypeStruct((N, 1), jnp.int32),
            jax.ShapeDtypeStruct((1, 1), jnp.float32),
        ),
        scratch_shapes=[
            pltpu.VMEM((1, LANES), jnp.float32),
            pltpu.VMEM((1, LANES), jnp.float32),
            pltpu.SMEM((1,), jnp.float32),
        ],
    )(x_flat, gw_pad)


def _worker_id():
    return lax.axis_index("c") * 16 + lax.axis_index("s")


@functools.partial(
    pl.kernel,
    out_type=(
        jax.ShapeDtypeStruct((N,), jnp.int32),
        jax.ShapeDtypeStruct((NW, 16), jnp.int32),
    ),
    mesh=_sc_mesh,
    compiler_params=_sc_params,
    scratch_types=[
        pltpu.VMEM((TPW,), jnp.int32),
        pltpu.VMEM((TPW,), jnp.int32),
        pltpu.VMEM((16,), jnp.int32),
    ],
)
def _sc_rank(top_hbm, lr_hbm, cnt_hbm, idx_v, lr_v, cnt_v):
    wid = _worker_id()
    base = wid * TPW
    pltpu.sync_copy(top_hbm.at[pl.ds(base, TPW)], idx_v)
    lanes16 = lax.iota(jnp.int32, 16)
    cnt_v[...] = jnp.zeros((16,), jnp.int32)
    for c in range(NCHUNK):
        idx = idx_v[pl.ds(c * 16, 16)]
        occ = jnp.zeros((16,), jnp.int32)
        hist = jnp.zeros((16,), jnp.int32)
        for e in range(E):
            mi = (idx == e).astype(jnp.int32)
            ce = plsc.cumsum(mi)
            occ = occ + (ce - 1) * mi
            hist = hist + jnp.where(lanes16 == e, jnp.max(ce), 0)
        old = plsc.load_gather(cnt_v, [idx])
        lr_v[pl.ds(c * 16, 16)] = old + occ
        cnt_v[...] = cnt_v[...] + hist
    pltpu.sync_copy(lr_v, lr_hbm.at[pl.ds(base, TPW)])
    pltpu.sync_copy(cnt_v, cnt_hbm.at[wid])


EG = 4
HDISP_ROWS = EG * C + NW


def _make_sc_dispatch(group):
    emit_meta = group == 0
    outs = [jax.ShapeDtypeStruct((HDISP_ROWS, D), jnp.float32)]
    if emit_meta:
        outs += [
            jax.ShapeDtypeStruct((N,), jnp.int32),
            jax.ShapeDtypeStruct((N,), jnp.int32),
        ]

    @functools.partial(
        pl.kernel,
        out_type=tuple(outs),
        mesh=_sc_mesh,
        compiler_params=_sc_params,
        scratch_types=[
            pltpu.VMEM((NW, 16), jnp.int32),
            pltpu.VMEM((16,), jnp.int32),
            pltpu.VMEM((TPW,), jnp.int32),
            pltpu.VMEM((TPW,), jnp.int32),
            pltpu.VMEM((TPW,), jnp.int32),
            pltpu.VMEM((TPW,), jnp.int32),
            pltpu.VMEM((64,), jnp.int32),
            pltpu.VMEM((64,), jnp.int32),
            pltpu.VMEM((64, D), jnp.float32),
            pltpu.SemaphoreType.DMA,
        ],
    )
    def dispatch_kernel(x_hbm, top_hbm, lr_hbm, cnt_hbm, disp_hbm, *rest):
        if emit_meta:
            slot_hbm, keep_hbm = rest[0], rest[1]
            rest = rest[2:]
        (ctb_v, off_v, idx_v, lr_v, slot_v, keep_v,
         sidx0_v, sidx1_v, xbuf_v, sem) = rest
        wid = _worker_id()
        base = wid * TPW
        pltpu.sync_copy(top_hbm.at[pl.ds(base, TPW)], idx_v)
        pltpu.sync_copy(lr_hbm.at[pl.ds(base, TPW)], lr_v)
        pltpu.sync_copy(cnt_hbm, ctb_v)
        off = jnp.zeros((16,), jnp.int32)
        for t in range(NW):
            row = ctb_v[t]
            off = off + jnp.where(t < wid, row, 0)
        off_v[...] = off
        for c in range(NCHUNK):
            idx = idx_v[pl.ds(c * 16, 16)]
            lrc = lr_v[pl.ds(c * 16, 16)]
            rank = lrc + plsc.load_gather(off_v, [idx])
            keep = (rank < C).astype(jnp.int32)
            slot = idx * C + jnp.minimum(rank, C - 1)
            if emit_meta:
                slot_v[pl.ds(c * 16, 16)] = slot
                keep_v[pl.ds(c * 16, 16)] = keep
            in_grp = jnp.logical_and(idx >= group * EG, idx < (group + 1) * EG)
            ok = jnp.logical_and(keep == 1, in_grp)
            sidx = jnp.where(ok, slot - group * EG * C, EG * C + wid)
            half = c // (NCHUNK // 2)
            part = c % (NCHUNK // 2)
            if half == 0:
                sidx0_v[pl.ds(part * 16, 16)] = sidx
            else:
                sidx1_v[pl.ds(part * 16, 16)] = sidx
        if emit_meta:
            pltpu.sync_copy(slot_v, slot_hbm.at[pl.ds(base, TPW)])
            pltpu.sync_copy(keep_v, keep_hbm.at[pl.ds(base, TPW)])
        pltpu.sync_copy(x_hbm.at[pl.ds(base, 64)], xbuf_v)
        pltpu.async_copy(xbuf_v, disp_hbm.at[sidx0_v], sem).wait()
        pltpu.sync_copy(x_hbm.at[pl.ds(base + 64, 64)], xbuf_v)
        pltpu.async_copy(xbuf_v, disp_hbm.at[sidx1_v], sem).wait()

    return dispatch_kernel


_sc_dispatch_lo = _make_sc_dispatch(0)
_sc_dispatch_hi = _make_sc_dispatch(1)


@functools.partial(
    pl.kernel,
    out_type=jax.ShapeDtypeStruct((N, D), jnp.float32),
    mesh=_sc_mesh,
    compiler_params=_sc_params,
    scratch_types=[
        pltpu.VMEM((TPW,), jnp.int32),
        pltpu.VMEM((64,), jnp.int32),
        pltpu.VMEM((64, D), jnp.float32),
        pltpu.SemaphoreType.DMA,
    ],
)
def _sc_combine(oute_hbm, slot_hbm, keep_hbm, y_hbm, keep_v, sidx_v, ybuf_v, sem):
    wid = _worker_id()
    base = wid * TPW
    pltpu.sync_copy(keep_hbm.at[pl.ds(base, TPW)], keep_v)
    lanes16 = lax.iota(jnp.int32, 16)
    zeros16 = jnp.zeros((16,), jnp.float32)
    for half in range(2):
        pltpu.sync_copy(slot_hbm.at[pl.ds(base + half * 64, 64)], sidx_v)
        pltpu.async_copy(oute_hbm.at[sidx_v], ybuf_v, sem).wait()
        kmin = keep_v[pl.ds(half * 64, 16)]
        for c in range(1, 4):
            kmin = jnp.minimum(kmin, keep_v[pl.ds(half * 64 + c * 16, 16)])
        anyd = jnp.min(kmin)

        @pl.when(anyd == 0)
        def _(half=half):
            def row_body(r, carry):
                kc = keep_v[pl.ds(half * 64 + (r // 16) * 16, 16)]
                kr = jnp.max(jnp.where(lanes16 == (r % 16), kc, 0))

                @pl.when(kr == 0)
                def _():
                    def k_body(k, c2):
                        ybuf_v[r, pl.ds(k * 16, 16)] = zeros16
                        return c2
                    lax.fori_loop(0, D // 16, k_body, 0)

                return carry
            lax.fori_loop(0, 64, row_body, 0)

        pltpu.sync_copy(ybuf_v, y_hbm.at[pl.ds(base + half * 64, 64)])


def _ffn_body(disp_ref, w13_ref, w2_ref, out_ref, dbf_ref):
    h = pl.program_id(1)

    @pl.when(h == 0)
    def _():
        dbf_ref[...] = disp_ref[...].astype(jnp.bfloat16)

    db = dbf_ref[...]
    wg = w13_ref[0, 0].astype(jnp.bfloat16)
    wu = w13_ref[0, 1].astype(jnp.bfloat16)
    g = lax.dot_general(db, wg, (((1,), (1,)), ((), ())),
                        preferred_element_type=jnp.float32)
    u = lax.dot_general(db, wu, (((1,), (1,)), ((), ())),
                        preferred_element_type=jnp.float32)
    swi = (g * jax.nn.sigmoid(g) * u).astype(jnp.bfloat16)
    w2b = w2_ref[0].astype(jnp.bfloat16)
    part = lax.dot_general(swi, w2b, (((1,), (1,)), ((), ())),
                           preferred_element_type=jnp.float32)

    @pl.when(h == 0)
    def _():
        out_ref[...] = part

    @pl.when(h != 0)
    def _():
        out_ref[...] += part


def _ffn_body_hi(disp_ref, w13_ref, w2_ref, oin_ref, out_ref, dbf_ref):
    del oin_ref
    _ffn_body(disp_ref, w13_ref, w2_ref, out_ref, dbf_ref)


def _ffn_lo(disp_lo, w13r, w2):
    return pl.pallas_call(
        _ffn_body,
        grid=(EG, NH),
        in_specs=[
            pl.BlockSpec((C, D), lambda e, h: (e, 0)),
            pl.BlockSpec((1, 2, HT, D), lambda e, h: (e, 0, h, 0)),
            pl.BlockSpec((1, D, HT), lambda e, h: (e, 0, h)),
        ],
        out_specs=pl.BlockSpec((C, D), lambda e, h: (e, 0)),
        out_shape=jax.ShapeDtypeStruct((EC, D), jnp.float32),
        scratch_shapes=[pltpu.VMEM((C, D), jnp.bfloat16)],
    )(disp_lo, w13r, w2)


def _ffn_hi(disp_hi, w13r, w2, oute_lo):
    return pl.pallas_call(
        _ffn_body_hi,
        grid=(EG, NH),
        in_specs=[
            pl.BlockSpec((C, D), lambda e, h: (e, 0)),
            pl.BlockSpec((1, 2, HT, D), lambda e, h: (e + EG, 0, h, 0)),
            pl.BlockSpec((1, D, HT), lambda e, h: (e + EG, 0, h)),
            pl.BlockSpec(memory_space=pl.ANY),
        ],
        out_specs=pl.BlockSpec((C, D), lambda e, h: (e + EG, 0)),
        out_shape=jax.ShapeDtypeStruct((EC, D), jnp.float32),
        scratch_shapes=[pltpu.VMEM((C, D), jnp.bfloat16)],
        input_output_aliases={3: 0},
    )(disp_hi, w13r, w2, oute_lo)


def kernel(x, gate_w, w13, w2):
    B_, T_, D_ = x.shape
    x_flat = x.reshape(N, D_)
    gw_pad = jnp.zeros((LANES, D), jnp.float32).at[:E].set(gate_w)
    top_col, aux = _router(x_flat, gw_pad)
    top_flat = top_col.reshape(N)
    lr, cnt = _sc_rank(top_flat)
    disp_lo, slot, keepi = _sc_dispatch_lo(x_flat, top_flat, lr, cnt)
    (disp_hi,) = _sc_dispatch_hi(x_flat, top_flat, lr, cnt)
    w13r = w13.reshape(E, 2, H, D)
    oute_lo = _ffn_lo(disp_lo, w13r, w2)
    out_e = _ffn_hi(disp_hi, w13r, w2, oute_lo)
    y = _sc_combine(out_e, slot, keepi)
    output = y.reshape(B_, T_, D_)
    return (output, aux.reshape(()), top_flat.reshape(B_, T_),
            (keepi != 0).reshape(B_, T_))


# speedup vs baseline: 1.1037x; 1.1037x over previous
import dataclasses
import functools

import jax
import jax.numpy as jnp
from jax import lax
from jax.experimental import pallas as pl
from jax.experimental.pallas import tpu as pltpu
from jax.experimental.pallas import tpu_sc as plsc

E = 8
AUX_W = 0.01
Z_W = 0.001
N = 4096
D = 1024
C = 640
LANES = 128
H = 2816
HT = 256
NH = H // HT
EC = E * C

NW = 32
TPW = N // NW
NCHUNK = TPW // 16
DISP_ROWS = EC + NW

_sc_mesh = plsc.VectorSubcoreMesh(core_axis_name="c", subcore_axis_name="s")

_sc_params = pltpu.CompilerParams()
if "needs_layout_passes" in pltpu.CompilerParams.__dataclass_fields__:
    _sc_params = dataclasses.replace(_sc_params, needs_layout_passes=False)


BT = 1024
NTB = N // BT


def _router_body(x_ref, gw_ref, idx_ref, cnt_ref, aux_ref, ps_ref, cn_ref, zq_ref):
    t = pl.program_id(0)
    x = x_ref[...]
    gw = gw_ref[...]
    logits = lax.dot_general(
        x.astype(jnp.bfloat16), gw.astype(jnp.bfloat16), (((1,), (1,)), ((), ())),
        preferred_element_type=jnp.float32,
    )
    lane = lax.broadcasted_iota(jnp.int32, (BT, LANES), 1)
    valid = lane < E
    lm = jnp.where(valid, logits, -1e30)
    m = jnp.max(lm, axis=1, keepdims=True)
    top = jnp.min(jnp.where(lm == m, lane, LANES), axis=1, keepdims=True)
    ex = jnp.where(valid, jnp.exp(lm - m), 0.0)
    denom = jnp.sum(ex, axis=1, keepdims=True)
    z = m + jnp.log(denom)
    zsq = jnp.sum(z * z)
    probs_sum = jnp.sum(ex / denom, axis=0, keepdims=True)
    one_hot = (lane == top).astype(jnp.float32)
    counts = jnp.sum(one_hot, axis=0, keepdims=True)
    idx_ref[...] = top
    grp = [jnp.sum(one_hot[g * TPW:(g + 1) * TPW], axis=0, keepdims=True)
           for g in range(BT // TPW)]
    cnt_ref[...] = jnp.concatenate(grp, axis=0).astype(jnp.int32)

    @pl.when(t == 0)
    def _():
        ps_ref[...] = jnp.zeros((1, LANES), jnp.float32)
        cn_ref[...] = jnp.zeros((1, LANES), jnp.float32)
        zq_ref[0] = 0.0

    ps_ref[...] += probs_sum
    cn_ref[...] += counts
    zq_ref[0] += zsq

    @pl.when(t == NTB - 1)
    def _():
        balance = jnp.sum(ps_ref[...] * cn_ref[...]) * (AUX_W * E / (N * N))
        aux_ref[0, 0] = balance + zq_ref[0] * (Z_W / N)


def _router(x_flat, gw_pad):
    return pl.pallas_call(
        _router_body,
        grid=(NTB,),
        in_specs=[
            pl.BlockSpec((BT, D), lambda t: (t, 0)),
            pl.BlockSpec((LANES, D), lambda t: (0, 0)),
        ],
        out_specs=(
            pl.BlockSpec((BT, 1), lambda t: (t, 0)),
            pl.BlockSpec((BT // TPW, LANES), lambda t: (t, 0)),
            pl.BlockSpec(memory_space=pltpu.SMEM),
        ),
        out_shape=(
            jax.ShapeDtypeStruct((N, 1), jnp.int32),
            jax.ShapeDtypeStruct((NW, LANES), jnp.int32),
            jax.ShapeDtypeStruct((1, 1), jnp.float32),
        ),
        scratch_shapes=[
            pltpu.VMEM((1, LANES), jnp.float32),
            pltpu.VMEM((1, LANES), jnp.float32),
            pltpu.SMEM((1,), jnp.float32),
        ],
    )(x_flat, gw_pad)


def _worker_id():
    return lax.axis_index("c") * 16 + lax.axis_index("s")


@functools.partial(
    pl.kernel,
    out_type=(
        jax.ShapeDtypeStruct((DISP_ROWS, D), jnp.float32),
        jax.ShapeDtypeStruct((N,), jnp.int32),
        jax.ShapeDtypeStruct((N,), jnp.int32),
    ),
    mesh=_sc_mesh,
    compiler_params=_sc_params,
    scratch_types=[
        pltpu.VMEM((NW, LANES), jnp.int32),
        pltpu.VMEM((16,), jnp.int32),
        pltpu.VMEM((TPW,), jnp.int32),
        pltpu.VMEM((TPW,), jnp.int32),
        pltpu.VMEM((TPW,), jnp.int32),
        pltpu.VMEM((64,), jnp.int32),
        pltpu.VMEM((64,), jnp.int32),
        pltpu.VMEM((64, D), jnp.float32),
        pltpu.SemaphoreType.DMA,
    ],
)
def _sc_dispatch(x_hbm, top_hbm, cnt_hbm,
                 disp_hbm, slot_hbm, keep_hbm,
                 ctb_v, cnt_v, idx_v, slot_v, keep_v,
                 sidx0_v, sidx1_v, xbuf_v, sem):
    wid = _worker_id()
    base = wid * TPW
    pltpu.sync_copy(top_hbm.at[pl.ds(base, TPW)], idx_v)
    pltpu.sync_copy(cnt_hbm, ctb_v)
    lanes16 = lax.iota(jnp.int32, 16)
    off = jnp.zeros((16,), jnp.int32)
    for t in range(NW):
        row = ctb_v[t, pl.ds(0, 16)]
        off = off + jnp.where(t < wid, row, 0)
    cnt_v[...] = off
    for c in range(NCHUNK):
        idx = idx_v[pl.ds(c * 16, 16)]
        occ = jnp.zeros((16,), jnp.int32)
        hist = jnp.zeros((16,), jnp.int32)
        for e in range(E):
            mi = (idx == e).astype(jnp.int32)
            ce = plsc.cumsum(mi)
            occ = occ + (ce - 1) * mi
            hist = hist + jnp.where(lanes16 == e, jnp.max(ce), 0)
        rank = plsc.load_gather(cnt_v, [idx]) + occ
        cnt_v[...] = cnt_v[...] + hist
        keep = (rank < C).astype(jnp.int32)
        slot = idx * C + jnp.minimum(rank, C - 1)
        slot_v[pl.ds(c * 16, 16)] = slot
        keep_v[pl.ds(c * 16, 16)] = keep
        sidx = jnp.where(keep == 1, slot, EC + wid)
        half = c // (NCHUNK // 2)
        part = c % (NCHUNK // 2)
        if half == 0:
            sidx0_v[pl.ds(part * 16, 16)] = sidx
        else:
            sidx1_v[pl.ds(part * 16, 16)] = sidx
    pltpu.sync_copy(slot_v, slot_hbm.at[pl.ds(base, TPW)])
    pltpu.sync_copy(keep_v, keep_hbm.at[pl.ds(base, TPW)])
    pltpu.sync_copy(x_hbm.at[pl.ds(base, 64)], xbuf_v)
    pltpu.async_copy(xbuf_v, disp_hbm.at[sidx0_v], sem).wait()
    pltpu.sync_copy(x_hbm.at[pl.ds(base + 64, 64)], xbuf_v)
    pltpu.async_copy(xbuf_v, disp_hbm.at[sidx1_v], sem).wait()


@functools.partial(
    pl.kernel,
    out_type=jax.ShapeDtypeStruct((N, D), jnp.float32),
    mesh=_sc_mesh,
    compiler_params=_sc_params,
    scratch_types=[
        pltpu.VMEM((TPW,), jnp.int32),
        pltpu.VMEM((64,), jnp.int32),
        pltpu.VMEM((64, D), jnp.float32),
        pltpu.SemaphoreType.DMA,
    ],
)
def _sc_combine(oute_hbm, slot_hbm, keep_hbm, y_hbm, keep_v, sidx_v, ybuf_v, sem):
    wid = _worker_id()
    base = wid * TPW
    pltpu.sync_copy(keep_hbm.at[pl.ds(base, TPW)], keep_v)
    lanes16 = lax.iota(jnp.int32, 16)
    zeros16 = jnp.zeros((16,), jnp.float32)
    for half in range(2):
        pltpu.sync_copy(slot_hbm.at[pl.ds(base + half * 64, 64)], sidx_v)
        pltpu.async_copy(oute_hbm.at[sidx_v], ybuf_v, sem).wait()
        kmin = keep_v[pl.ds(half * 64, 16)]
        for c in range(1, 4):
            kmin = jnp.minimum(kmin, keep_v[pl.ds(half * 64 + c * 16, 16)])
        anyd = jnp.min(kmin)

        @pl.when(anyd == 0)
        def _(half=half):
            def row_body(r, carry):
                kc = keep_v[pl.ds(half * 64 + (r // 16) * 16, 16)]
                kr = jnp.max(jnp.where(lanes16 == (r % 16), kc, 0))

                @pl.when(kr == 0)
                def _():
                    def k_body(k, c2):
                        ybuf_v[r, pl.ds(k * 16, 16)] = zeros16
                        return c2
                    lax.fori_loop(0, D // 16, k_body, 0)

                return carry
            lax.fori_loop(0, 64, row_body, 0)

        pltpu.sync_copy(ybuf_v, y_hbm.at[pl.ds(base + half * 64, 64)])


def _ffn_body(disp_ref, w13_ref, w2_ref, out_ref, dbf_ref):
    h = pl.program_id(1)

    @pl.when(h == 0)
    def _():
        dbf_ref[...] = disp_ref[...].astype(jnp.bfloat16)

    db = dbf_ref[...]
    wg = w13_ref[0, 0].astype(jnp.bfloat16)
    wu = w13_ref[0, 1].astype(jnp.bfloat16)
    g = lax.dot_general(db, wg, (((1,), (1,)), ((), ())),
                        preferred_element_type=jnp.float32)
    u = lax.dot_general(db, wu, (((1,), (1,)), ((), ())),
                        preferred_element_type=jnp.float32)
    swi = (g * jax.nn.sigmoid(g) * u).astype(jnp.bfloat16)
    w2b = w2_ref[0].astype(jnp.bfloat16)
    part = lax.dot_general(swi, w2b, (((1,), (1,)), ((), ())),
                           preferred_element_type=jnp.float32)

    @pl.when(h == 0)
    def _():
        out_ref[...] = part

    @pl.when(h != 0)
    def _():
        out_ref[...] += part


def _ffn(disp, w13r, w2):
    return pl.pallas_call(
        _ffn_body,
        grid=(E, NH),
        in_specs=[
            pl.BlockSpec((C, D), lambda e, h: (e, 0)),
            pl.BlockSpec((1, 2, HT, D), lambda e, h: (e, 0, h, 0)),
            pl.BlockSpec((1, D, HT), lambda e, h: (e, 0, h)),
        ],
        out_specs=pl.BlockSpec((C, D), lambda e, h: (e, 0)),
        out_shape=jax.ShapeDtypeStruct((EC, D), jnp.float32),
        scratch_shapes=[pltpu.VMEM((C, D), jnp.bfloat16)],
    )(disp, w13r, w2)


def kernel(x, gate_w, w13, w2):
    B_, T_, D_ = x.shape
    x_flat = x.reshape(N, D_)
    gw_pad = jnp.zeros((LANES, D), jnp.float32).at[:E].set(gate_w)
    top_col, cnt, aux = _router(x_flat, gw_pad)
    top_flat = top_col.reshape(N)
    disp, slot, keepi = _sc_dispatch(x_flat, top_flat, cnt)
    w13r = w13.reshape(E, 2, H, D)
    out_e = _ffn(disp, w13r, w2)
    y = _sc_combine(out_e, slot, keepi)
    output = y.reshape(B_, T_, D_)
    return (output, aux.reshape(()), top_flat.reshape(B_, T_),
            (keepi != 0).reshape(B_, T_))
